# baseline (speedup 1.0000x reference)
import jax
import jax.numpy as jnp
from jax.experimental import pallas as pl
from jax.experimental.pallas import tpu as pltpu

_N = 64
_E = _N * _N
_HID = 256
_AM = 5
_NL = 4
_SCALE = 10.0

_INTERPRET = False
_PREC = jax.lax.Precision.HIGHEST
_EC = 2048
_NCH = _E // _EC


def _precise_div(a, b):
    r = 1.0 / b
    r = r * (2.0 - b * r)
    r = r * (2.0 - b * r)
    q = a * r
    return q + (a - q * b) * r


def _precise_sqrt(x):
    s0 = jnp.sqrt(x)
    s1 = 0.5 * (s0 + _precise_div(x, s0))
    return jnp.where(x > 0.0, s1, 0.0)


def _tanh(x):
    cap = 7.90531110763549805
    xc = jnp.clip(x, -cap, cap)
    x2 = xc * xc
    num = xc * (4.89352455891786e-03
                + x2 * (6.37261928875436e-04
                + x2 * (1.48572235717979e-05
                + x2 * (5.12229709037114e-08
                + x2 * (-8.60467152213735e-11
                + x2 * (2.00018790482477e-13
                + x2 * (-2.76076847742355e-16)))))))
    den = (4.89352518554385e-03
           + x2 * (2.26843463243900e-03
           + x2 * (1.18534705686654e-04
           + x2 * 1.19825839466702e-06)))
    r = _precise_div(num, den)
    return jnp.where(jnp.abs(x) < 0.0004, x, r)


def _sigmoid(x):
    return 0.5 + 0.5 * _tanh(0.5 * x)


def _silu(v):
    return v * _sigmoid(v)


def _dot(a, b):
    f32 = jnp.float32
    bf = jnp.bfloat16
    ah = a.astype(bf)
    al = (a - ah.astype(f32)).astype(bf)
    bh = b.astype(bf)
    bl = (b - bh.astype(f32)).astype(bf)
    d = lambda u, v: jnp.dot(u, v, preferred_element_type=f32)
    return d(ah, bh) + d(ah, bl) + d(al, bh)


def _dott(a, b):
    return jax.lax.dot_general(a, b, (((0,), (0,)), ((), ())),
                               preferred_element_type=jnp.float32,
                               precision=_PREC)


def _split3(v):
    bf16 = jnp.bfloat16
    f32 = jnp.float32
    v1 = v.astype(bf16)
    r1 = v - v1.astype(f32)
    v2 = r1.astype(bf16)
    r2 = r1 - v2.astype(f32)
    v3 = r2.astype(bf16)
    return v1, v2, v3


def _exact_gather(Gb, parts):
    g1 = jnp.dot(Gb, parts[0], preferred_element_type=jnp.float32)
    g2 = jnp.dot(Gb, parts[1], preferred_element_type=jnp.float32)
    g3 = jnp.dot(Gb, parts[2], preferred_element_type=jnp.float32)
    return (g1 + g2) + g3


def _egnn_kernel(xin_ref, hin_ref, t_ref, ii_ref, ij_ref,
                 win_ref, bin_ref,
                 Wfe_ref, b1_ref,
                 Wx2_ref, bx2_ref, wx3_ref,
                 We2_ref, be2_ref, wat_ref, bat_ref,
                 Wh1_ref, bh1_ref, Wh2_ref, bh2_ref,
                 wout_ref, bout_ref, out_ref, ea_ref):
    f32 = jnp.float32
    x_in = xin_ref[0]
    h_raw = hin_ref[0]
    tt = t_ref[0]

    lanes = jax.lax.broadcasted_iota(jnp.int32, (_EC, _N), 1)

    def masks(c):
        ic = ii_ref[0, 0, pl.ds(c * _EC, _EC)]
        jc = ij_ref[0, 0, pl.ds(c * _EC, _EC)]
        Gi = (lanes == ic[:, None]).astype(jnp.bfloat16)
        Gj = (lanes == jc[:, None]).astype(jnp.bfloat16)
        return Gi, Gj

    def ea_body(c, carry):
        Gi, Gj = masks(c)
        xp = _split3(x_in)
        diff = _exact_gather(Gi, xp) - _exact_gather(Gj, xp)
        d = _precise_sqrt(jnp.sum(diff * diff, axis=1, keepdims=True))
        ea_ref[pl.ds(c * _EC, _EC), :] = d
        return carry

    jax.lax.fori_loop(0, _NCH, ea_body, 0)

    h0 = _dot(jnp.concatenate([h_raw, tt], axis=1), win_ref[...]) + bin_ref[...]

    def layer_body(l, xh):
        x, h = xh
        xp = _split3(x)
        hp = _split3(h)

        def chunk_body(c, agg):
            Gi, Gj = masks(c)
            diff = _exact_gather(Gi, xp) - _exact_gather(Gj, xp)
            d2 = jnp.sum(diff * diff, axis=1, keepdims=True)
            d = _precise_sqrt(d2)
            dd = d * d
            ea = ea_ref[pl.ds(c * _EC, _EC), :]
            h_i = _exact_gather(Gi, hp)
            h_j = _exact_gather(Gj, hp)
            feat = jnp.concatenate([h_i, h_j, dd, ea], axis=1)
            pre = _dot(feat, Wfe_ref[l]) + b1_ref[l]
            px = pre[:, :_HID]
            pe = pre[:, _HID:]

            xm = _silu(px)
            xm = _silu(_dot(xm, Wx2_ref[l]) + bx2_ref[l])
            xw = _tanh(_dot(xm, wx3_ref[l]))
            xv = _precise_div(diff, d + 1.0) * xw * _SCALE

            m = _silu(pe)
            m = _silu(_dot(m, We2_ref[l]) + be2_ref[l])
            e = _sigmoid(_dot(m, wat_ref[l]) + bat_ref[l])
            em = e * m

            return agg + _dott(Gi.astype(f32),
                               jnp.concatenate([em, xv], axis=1))

        agg = jax.lax.fori_loop(0, _NCH, chunk_body,
                                jnp.zeros((_N, _HID + 3), dtype=f32))
        em_agg = agg[:, :_HID]
        x = x + agg[:, _HID:]
        hh = _silu(_dot(jnp.concatenate([h, em_agg], axis=1), Wh1_ref[l])
                   + bh1_ref[l])
        hh = _dot(hh, Wh2_ref[l]) + bh2_ref[l]
        return x, h + hh

    x, h = jax.lax.fori_loop(0, _NL, layer_body, (x_in, h0))

    xo = x - x_in
    xo = xo - jnp.mean(xo, axis=0, keepdims=True)
    ho = _dot(h, wout_ref[...]) + bout_ref[...]
    out_ref[0] = jnp.concatenate([xo, ho[:, :_AM]], axis=1)


def _pack_params(params):
    win = params['dense_in'][0]
    bin_ = params['dense_in'][1][None, :]
    Wfe, b1 = [], []
    Wx2, bx2, wx3 = [], [], []
    We2, be2, wat, bat = [], [], [], []
    Wh1l, bh1, Wh2, bh2 = [], [], [], []
    for blk in params['blocks']:
        Wx1, bx1_, Wx2_, bx2_, wx3_ = blk['dense_x']
        We1, be1_, We2_, be2_ = blk['dense_e']
        wat_, bat_ = blk['e_attention']
        Wh1, bh1_, Wh2_, bh2_ = blk['dense_h']
        Wfe.append(jnp.concatenate([Wx1, We1], axis=1))
        b1.append(jnp.concatenate([bx1_, be1_])[None, :])
        Wx2.append(Wx2_)
        bx2.append(bx2_[None, :])
        wx3.append(wx3_)
        We2.append(We2_)
        be2.append(be2_[None, :])
        wat.append(wat_)
        bat.append(bat_[None, :])
        Wh1l.append(Wh1)
        bh1.append(bh1_[None, :])
        Wh2.append(Wh2_)
        bh2.append(bh2_[None, :])
    stk = jnp.stack
    return (win, bin_, stk(Wfe), stk(b1),
            stk(Wx2), stk(bx2), stk(wx3),
            stk(We2), stk(be2), stk(wat), stk(bat),
            stk(Wh1l), stk(bh1), stk(Wh2), stk(bh2),
            params['dense_out'][0], params['dense_out'][1][None, :])


def kernel(x_in, h_in, t, edge_indices, node_mask, edge_mask, params):
    del node_mask, edge_mask
    B = x_in.shape[0]
    idx = edge_indices.astype(jnp.int32)
    idx_i = idx[..., 0].reshape(B, 1, _E)
    idx_j = idx[..., 1].reshape(B, 1, _E)
    packed = _pack_params(params)

    def bspec(shape, mapped_leading=False):
        nd = len(shape)
        if mapped_leading:
            return pl.BlockSpec((1,) + shape[1:],
                                lambda b: (b,) + (0,) * (nd - 1))
        return pl.BlockSpec(shape, lambda b, _nd=nd: (0,) * _nd)

    in_specs = [
        bspec(x_in.shape, True),
        bspec(h_in.shape, True),
        bspec(t.shape, True),
        bspec(idx_i.shape, True),
        bspec(idx_j.shape, True),
    ] + [bspec(p.shape) for p in packed]

    out = pl.pallas_call(
        _egnn_kernel,
        grid=(B,),
        in_specs=in_specs,
        out_specs=pl.BlockSpec((1, _N, 3 + _AM), lambda b: (b, 0, 0)),
        out_shape=jax.ShapeDtypeStruct((B, _N, 3 + _AM), jnp.float32),
        scratch_shapes=[pltpu.VMEM((_E, 1), jnp.float32)],
        compiler_params=pltpu.CompilerParams(
            dimension_semantics=("parallel",)),
        interpret=_INTERPRET,
    )(x_in, h_in, t, idx_i, idx_j, *packed)
    return out
